# baseline (speedup 1.0000x reference)
import functools

import jax
import jax.numpy as jnp
from jax import lax
from jax.experimental import pallas as pl
from jax.experimental.pallas import tpu as pltpu
from jax.experimental.pallas import tpu_sc as plsc

_BATCH = 4
_SEQ = 4096
_DIM = 1024
_NC = 2
_NS = 16
_NW = _NC * _NS
_SPW = _SEQ // _NW
_CHUNK = 16
_NCHUNK = _SPW // _CHUNK
_LANES = 16
_GRP = _BATCH * _SPW // _LANES

_mesh = plsc.VectorSubcoreMesh(core_axis_name="c", subcore_axis_name="s")


@functools.partial(
    pl.kernel,
    mesh=_mesh,
    compiler_params=pltpu.CompilerParams(needs_layout_passes=False),
    out_type=jax.ShapeDtypeStruct((_BATCH, _SEQ, _DIM), jnp.float32),
    scratch_types=[
        pltpu.VMEM((_BATCH, _SPW), jnp.int32),
        pltpu.VMEM((_SPW,), jnp.int32),
        pltpu.VMEM((_CHUNK, _DIM), jnp.float32),
        pltpu.VMEM((_CHUNK, _DIM), jnp.float32),
        pltpu.VMEM((_LANES, _DIM), jnp.float32),
        pltpu.VMEM((_GRP,), jnp.int32),
        pltpu.SemaphoreType.DMA,
        pltpu.SemaphoreType.DMA,
        pltpu.SemaphoreType.DMA,
        pltpu.SemaphoreType.DMA,
        pltpu.SemaphoreType.DMA,
        pltpu.SemaphoreType.DMA,
    ],
)
def _sc_embed(inp_hbm, table_hbm, out_hbm, inp_v, rows_v, buf0, buf1, fixbuf,
              flags_v, rsem0, rsem1, wsem0, wsem1, isem, fsem):
    wid = lax.axis_index("s") * _NC + lax.axis_index("c")
    s_base = wid * _SPW

    lanes = lax.iota(jnp.int32, _LANES)
    for k in range(_SPW // _LANES):
        rows_v[pl.ds(k * _LANES, _LANES)] = lanes + (s_base + k * _LANES + 1)

    bufs = (buf0, buf1)
    rsems = (rsem0, rsem1)
    wsems = (wsem0, wsem1)
    read_h = [None, None]
    write_h = [[], []]

    read_h[0] = pltpu.async_copy(
        table_hbm.at[rows_v.at[pl.ds(0, _CHUNK)]], bufs[0], rsems[0])
    inp_h = pltpu.async_copy(inp_hbm.at[:, pl.ds(s_base, _SPW)], inp_v, isem)

    for c in range(_NCHUNK):
        cur = c & 1
        nxt = 1 - cur
        if c + 1 < _NCHUNK:
            for h in write_h[nxt]:
                h.wait()
            write_h[nxt] = []
            read_h[nxt] = pltpu.async_copy(
                table_hbm.at[rows_v.at[pl.ds((c + 1) * _CHUNK, _CHUNK)]],
                bufs[nxt], rsems[nxt])
        read_h[cur].wait()
        for b in range(_BATCH):
            write_h[cur].append(pltpu.async_copy(
                bufs[cur],
                out_hbm.at[b, pl.ds(s_base + c * _CHUNK, _CHUNK)],
                wsems[cur]))

    inp_h.wait()
    for h in range(_GRP // _LANES):
        acc = jnp.zeros((_LANES,), jnp.int32)
        for t in range(_LANES):
            flat = lanes * _LANES + (h * _LANES * _LANES + t)
            w = plsc.load_gather(inp_v, [flat >> 7, flat & (_SPW - 1)])
            acc = acc | jnp.where(w == 0, 1, 0)
        flags_v[pl.ds(h * _LANES, _LANES)] = acc

    for side in (0, 1):
        for h in write_h[side]:
            h.wait()

    for j in range(_GRP):
        b = j // (_SPW // _LANES)
        k = j % (_SPW // _LANES)
        fv = flags_v[pl.ds((j // _LANES) * _LANES, _LANES)]
        has_pad = fv[j % _LANES] > 0

        @pl.when(has_pad)
        def _fix(b=b, k=k):
            v = inp_v[b, pl.ds(k * _LANES, _LANES)]
            pos = lanes + (s_base + k * _LANES + 1)
            idx = jnp.where(v != 0, pos, 0)
            pltpu.async_copy(table_hbm.at[idx], fixbuf, fsem).wait()
            pltpu.async_copy(
                fixbuf,
                out_hbm.at[b, pl.ds(s_base + k * _LANES, _LANES)],
                fsem).wait()


def kernel(input, weights):
    return _sc_embed(input.astype(jnp.int32), weights)


# speedup vs baseline: 1.0489x; 1.0489x over previous
import functools

import jax
import jax.numpy as jnp
from jax import lax
from jax.experimental import pallas as pl
from jax.experimental.pallas import tpu as pltpu
from jax.experimental.pallas import tpu_sc as plsc

_BATCH = 4
_SEQ = 4096
_DIM = 1024
_NC = 2
_NS = 16
_NW = _NC * _NS
_SPW = _SEQ // _NW
_CHUNK = 48
_CHUNKS = ((0, 48), (48, 48), (96, 32))
_LANES = 16
_GRP = _BATCH * _SPW // _LANES

_mesh = plsc.VectorSubcoreMesh(core_axis_name="c", subcore_axis_name="s")


@functools.partial(
    pl.kernel,
    mesh=_mesh,
    compiler_params=pltpu.CompilerParams(needs_layout_passes=False),
    out_type=jax.ShapeDtypeStruct((_BATCH, _SEQ, _DIM), jnp.float32),
    scratch_types=[
        pltpu.VMEM((_BATCH, _SPW), jnp.int32),
        pltpu.VMEM((_SPW,), jnp.int32),
        pltpu.VMEM((_CHUNK, _DIM), jnp.float32),
        pltpu.VMEM((_CHUNK, _DIM), jnp.float32),
        pltpu.VMEM((_LANES, _DIM), jnp.float32),
        pltpu.VMEM((_GRP,), jnp.int32),
        pltpu.SemaphoreType.DMA,
        pltpu.SemaphoreType.DMA,
        pltpu.SemaphoreType.DMA,
        pltpu.SemaphoreType.DMA,
        pltpu.SemaphoreType.DMA,
        pltpu.SemaphoreType.DMA,
    ],
)
def _sc_embed(inp_hbm, table_hbm, out_hbm, inp_v, rows_v, buf0, buf1, fixbuf,
              flags_v, rsem0, rsem1, wsem0, wsem1, isem, fsem):
    wid = lax.axis_index("s") * _NC + lax.axis_index("c")
    s_base = wid * _SPW

    lanes = lax.iota(jnp.int32, _LANES)
    for k in range(_SPW // _LANES):
        rows_v[pl.ds(k * _LANES, _LANES)] = lanes + (s_base + k * _LANES + 1)

    bufs = (buf0, buf1)
    rsems = (rsem0, rsem1)
    wsems = (wsem0, wsem1)
    read_h = [None, None]
    write_h = [[], []]

    read_h[0] = pltpu.async_copy(
        table_hbm.at[rows_v.at[pl.ds(_CHUNKS[0][0], _CHUNKS[0][1])]],
        bufs[0].at[pl.ds(0, _CHUNKS[0][1])], rsems[0])
    inp_h = pltpu.async_copy(inp_hbm.at[:, pl.ds(s_base, _SPW)], inp_v, isem)

    for c, (off, n) in enumerate(_CHUNKS):
        cur = c & 1
        nxt = 1 - cur
        if c + 1 < len(_CHUNKS):
            for h in write_h[nxt]:
                h.wait()
            write_h[nxt] = []
            noff, nn = _CHUNKS[c + 1]
            read_h[nxt] = pltpu.async_copy(
                table_hbm.at[rows_v.at[pl.ds(noff, nn)]],
                bufs[nxt].at[pl.ds(0, nn)], rsems[nxt])
        read_h[cur].wait()
        for b in range(_BATCH):
            write_h[cur].append(pltpu.async_copy(
                bufs[cur].at[pl.ds(0, n)],
                out_hbm.at[b, pl.ds(s_base + off, n)],
                wsems[cur]))

    inp_h.wait()
    for h in range(_GRP // _LANES):
        acc = jnp.zeros((_LANES,), jnp.int32)
        for t in range(_LANES):
            flat = lanes * _LANES + (h * _LANES * _LANES + t)
            w = plsc.load_gather(inp_v, [flat >> 7, flat & (_SPW - 1)])
            acc = acc | jnp.where(w == 0, 1, 0)
        flags_v[pl.ds(h * _LANES, _LANES)] = acc

    for side in (0, 1):
        for h in write_h[side]:
            h.wait()

    for j in range(_GRP):
        b = j // (_SPW // _LANES)
        k = j % (_SPW // _LANES)
        fv = flags_v[pl.ds((j // _LANES) * _LANES, _LANES)]
        has_pad = fv[j % _LANES] > 0

        @pl.when(has_pad)
        def _fix(b=b, k=k):
            v = inp_v[b, pl.ds(k * _LANES, _LANES)]
            pos = lanes + (s_base + k * _LANES + 1)
            idx = jnp.where(v != 0, pos, 0)
            pltpu.async_copy(table_hbm.at[idx], fixbuf, fsem).wait()
            pltpu.async_copy(
                fixbuf,
                out_hbm.at[b, pl.ds(s_base + k * _LANES, _LANES)],
                fsem).wait()


def kernel(input, weights):
    return _sc_embed(input.astype(jnp.int32), weights)


# speedup vs baseline: 1.0562x; 1.0069x over previous
import functools

import jax
import jax.numpy as jnp
from jax import lax
from jax.experimental import pallas as pl
from jax.experimental.pallas import tpu as pltpu
from jax.experimental.pallas import tpu_sc as plsc

_BATCH = 4
_SEQ = 4096
_DIM = 1024
_NC = 2
_NS = 16
_NW = _NC * _NS
_SPW = _SEQ // _NW
_CHUNKS = ((0, 64), (64, 56), (120, 8))
_LANES = 16
_GRP = _BATCH * _SPW // _LANES

_mesh = plsc.VectorSubcoreMesh(core_axis_name="c", subcore_axis_name="s")


@functools.partial(
    pl.kernel,
    mesh=_mesh,
    compiler_params=pltpu.CompilerParams(needs_layout_passes=False),
    out_type=jax.ShapeDtypeStruct((_BATCH, _SEQ, _DIM), jnp.float32),
    scratch_types=[
        pltpu.VMEM((_BATCH, _SPW), jnp.int32),
        pltpu.VMEM((_SPW,), jnp.int32),
        pltpu.VMEM((64, _DIM), jnp.float32),
        pltpu.VMEM((56, _DIM), jnp.float32),
        pltpu.VMEM((_GRP,), jnp.int32),
        pltpu.SemaphoreType.DMA,
        pltpu.SemaphoreType.DMA,
        pltpu.SemaphoreType.DMA,
        pltpu.SemaphoreType.DMA,
        pltpu.SemaphoreType.DMA,
        pltpu.SemaphoreType.DMA,
    ],
)
def _sc_embed(inp_hbm, table_hbm, out_hbm, inp_v, rows_v, buf0, buf1,
              flags_v, rsem0, rsem1, wsem0, wsem1, isem, fsem):
    wid = lax.axis_index("s") * _NC + lax.axis_index("c")
    s_base = wid * _SPW

    lanes = lax.iota(jnp.int32, _LANES)
    for k in range(_SPW // _LANES):
        rows_v[pl.ds(k * _LANES, _LANES)] = lanes + (s_base + k * _LANES + 1)

    bufs = (buf0, buf1)
    rsems = (rsem0, rsem1)
    wsems = (wsem0, wsem1)
    read_h = [None, None]
    write_h = [[], []]

    read_h[0] = pltpu.async_copy(
        table_hbm.at[rows_v.at[pl.ds(_CHUNKS[0][0], _CHUNKS[0][1])]],
        bufs[0].at[pl.ds(0, _CHUNKS[0][1])], rsems[0])
    inp_h = pltpu.async_copy(inp_hbm.at[:, pl.ds(s_base, _SPW)], inp_v, isem)

    for c, (off, n) in enumerate(_CHUNKS):
        cur = c & 1
        nxt = 1 - cur
        if c + 1 < len(_CHUNKS):
            for h in write_h[nxt]:
                h.wait()
            write_h[nxt] = []
            noff, nn = _CHUNKS[c + 1]
            read_h[nxt] = pltpu.async_copy(
                table_hbm.at[rows_v.at[pl.ds(noff, nn)]],
                bufs[nxt].at[pl.ds(0, nn)], rsems[nxt])
        read_h[cur].wait()
        for b in range(_BATCH):
            write_h[cur].append(pltpu.async_copy(
                bufs[cur].at[pl.ds(0, n)],
                out_hbm.at[b, pl.ds(s_base + off, n)],
                wsems[cur]))

    inp_h.wait()
    for h in range(_GRP // _LANES):
        acc = jnp.zeros((_LANES,), jnp.int32)
        for t in range(_LANES):
            flat = lanes * _LANES + (h * _LANES * _LANES + t)
            w = plsc.load_gather(inp_v, [flat >> 7, flat & (_SPW - 1)])
            acc = acc | jnp.where(w == 0, 1, 0)
        flags_v[pl.ds(h * _LANES, _LANES)] = acc

    for side in (0, 1):
        for h in write_h[side]:
            h.wait()

    fixbuf = buf0.at[pl.ds(0, _LANES)]
    for j in range(_GRP):
        b = j // (_SPW // _LANES)
        k = j % (_SPW // _LANES)
        fv = flags_v[pl.ds((j // _LANES) * _LANES, _LANES)]
        has_pad = fv[j % _LANES] > 0

        @pl.when(has_pad)
        def _fix(b=b, k=k):
            v = inp_v[b, pl.ds(k * _LANES, _LANES)]
            pos = lanes + (s_base + k * _LANES + 1)
            idx = jnp.where(v != 0, pos, 0)
            pltpu.async_copy(table_hbm.at[idx], fixbuf, fsem).wait()
            pltpu.async_copy(
                fixbuf,
                out_hbm.at[b, pl.ds(s_base + k * _LANES, _LANES)],
                fsem).wait()


def kernel(input, weights):
    return _sc_embed(input.astype(jnp.int32), weights)
